# baseline (speedup 1.0000x reference)
import functools

import jax
import jax.numpy as jnp
from jax import lax
from jax.experimental import pallas as pl
from jax.experimental.pallas import tpu as pltpu
from jax.experimental.pallas import tpu_sc as plsc

_VOCAB = 10000
_EMBED = 16
_BATCH = 16384
_MAXLEN = 500

_NC = 2
_NS = 16
_NW = _NC * _NS
_ROWS_PER_W = _BATCH // _NW
_C = 8
_NCHUNK = _ROWS_PER_W // _C
_NFULL = _MAXLEN // 16
_VPAD = _VOCAB + 16
_NPAIR = _EMBED // 2


def _sc_pool(idx_hbm, tbl_hbm, part_hbm, idx_a, idx_b, tbl_v, acc_v,
             sem_a, sem_b):
    wid = lax.axis_index("s") * _NC + lax.axis_index("c")
    row0 = wid * _ROWS_PER_W

    pltpu.sync_copy(tbl_hbm, tbl_v)

    cvecs = [jnp.full((16,), c, jnp.int32) for c in range(_NPAIR)]
    himask = jnp.full((16,), -65536, jnp.int32)
    zero = jnp.zeros((16,), jnp.float32)

    def load_idx(ci, buf, sem):
        r0 = row0 + ci * _C
        pltpu.async_copy(idx_hbm.at[pl.ds(r0 * _MAXLEN, _C * _MAXLEN)], buf,
                         sem)

    def drain_idx(ci, buf, sem):
        r0 = row0 + ci * _C
        pltpu.make_async_copy(idx_hbm.at[pl.ds(r0 * _MAXLEN, _C * _MAXLEN)],
                              buf, sem).wait()

    lane = lax.iota(jnp.int32, 16)
    tailkeep = lane >= (16 - (_MAXLEN - _NFULL * 16))

    def compute(ci, buf):
        r0 = row0 + ci * _C
        for r in range(_C):
            def gather_add(idx16, accs, keep=None):
                out = list(accs)
                base = jnp.left_shift(idx16, 3)
                for c in range(_NPAIR):
                    w = plsc.load_gather(tbl_v, [base + c] if c else [base])
                    lo = plsc.bitcast(jnp.left_shift(w, 16), jnp.float32)
                    hi = plsc.bitcast(jnp.bitwise_and(w, himask), jnp.float32)
                    if keep is not None:
                        lo = jnp.where(keep, lo, 0.0)
                        hi = jnp.where(keep, hi, 0.0)
                    out[2 * c] = out[2 * c] + lo
                    out[2 * c + 1] = out[2 * c + 1] + hi
                return tuple(out)

            def step(t, accs):
                return gather_add(buf[pl.ds(r * _MAXLEN + t * 16, 16)], accs)

            accs = lax.fori_loop(0, _NFULL, step, (zero,) * _EMBED)
            tail = buf[pl.ds(r * _MAXLEN + _MAXLEN - 16, 16)]
            accs = gather_add(tail, accs, keep=tailkeep)
            for c2 in range(_EMBED):
                acc_v[r, pl.ds(c2 * 16, 16)] = accs[c2]
        pltpu.sync_copy(acc_v, part_hbm.at[pl.ds(r0, _C)])

    pltpu.sync_copy(idx_hbm.at[pl.ds(row0 * _MAXLEN, _C * _MAXLEN)], idx_a)

    def pair_body(k, carry):
        ci0 = 2 * k
        ci1 = 2 * k + 1
        load_idx(ci1, idx_b, sem_b)
        compute(ci0, idx_a)
        drain_idx(ci1, idx_b, sem_b)

        @pl.when(k < _NCHUNK // 2 - 1)
        def _():
            load_idx(ci1 + 1, idx_a, sem_a)

        compute(ci1, idx_b)

        @pl.when(k < _NCHUNK // 2 - 1)
        def _():
            drain_idx(ci1 + 1, idx_a, sem_a)

        return carry

    lax.fori_loop(0, _NCHUNK // 2, pair_body, 0)


def _make_sc_pool():
    mesh = plsc.VectorSubcoreMesh(core_axis_name="c", subcore_axis_name="s")
    return functools.partial(
        pl.kernel,
        mesh=mesh,
        compiler_params=pltpu.CompilerParams(use_tc_tiling_on_sc=False, needs_layout_passes=False),
        out_type=jax.ShapeDtypeStruct((_BATCH, _EMBED * 16), jnp.float32),
        scratch_types=[
            pltpu.VMEM((_C * _MAXLEN,), jnp.int32),
            pltpu.VMEM((_C * _MAXLEN,), jnp.int32),
            pltpu.VMEM((_NPAIR * _VPAD,), jnp.int32),
            pltpu.VMEM((_C, _EMBED * 16), jnp.float32),
            pltpu.SemaphoreType.DMA,
            pltpu.SemaphoreType.DMA,
        ],
    )(_sc_pool)


def _mlp_body(part_ref, fold_ref, w1_ref, b1_ref, w2r_ref, b2_ref, out_ref):
    pooled = jnp.dot(part_ref[...], fold_ref[...],
                     preferred_element_type=jnp.float32) * (1.0 / _MAXLEN)
    h = jnp.dot(pooled, w1_ref[...], preferred_element_type=jnp.float32)
    h = jnp.maximum(h + b1_ref[...], 0.0)
    z = jnp.sum(h * w2r_ref[...], axis=1, keepdims=True) + b2_ref[...]
    out_ref[...] = 1.0 / (1.0 + jnp.exp(-z))


def _mlp(part, fold, W1, b1, W2, b2):
    blk = 2048
    grid = (_BATCH // blk,)
    return pl.pallas_call(
        _mlp_body,
        grid=grid,
        in_specs=[
            pl.BlockSpec((blk, _EMBED * 16), lambda i: (i, 0)),
            pl.BlockSpec((_EMBED * 16, _EMBED), lambda i: (0, 0)),
            pl.BlockSpec((_EMBED, _EMBED), lambda i: (0, 0)),
            pl.BlockSpec((1, _EMBED), lambda i: (0, 0)),
            pl.BlockSpec((1, _EMBED), lambda i: (0, 0)),
            pl.BlockSpec((1, 1), lambda i: (0, 0)),
        ],
        out_specs=pl.BlockSpec((blk, 1), lambda i: (i, 0)),
        out_shape=jax.ShapeDtypeStruct((_BATCH, 1), jnp.float32),
    )(part, fold, W1, b1.reshape(1, _EMBED), W2.reshape(1, _EMBED),
      b2.reshape(1, 1))


def kernel(inputs, table, W1, b1, W2, b2):
    idx = inputs.astype(jnp.int32).reshape(-1)
    tpad = jnp.concatenate(
        [table, jnp.zeros((_VPAD - _VOCAB, _EMBED), jnp.float32)], axis=0)
    bits = jax.lax.bitcast_convert_type(
        tpad.astype(jnp.bfloat16), jnp.uint16).astype(jnp.uint32)
    packed = bits[:, 0::2] | (bits[:, 1::2] << 16)
    tblT = jax.lax.bitcast_convert_type(packed, jnp.int32).reshape(-1)
    part = _make_sc_pool()(idx, tblT)
    fold = jnp.repeat(jnp.eye(_EMBED, dtype=jnp.float32), 16, axis=0)
    return _mlp(part, fold, W1, b1, W2, b2)


# speedup vs baseline: 1.0584x; 1.0584x over previous
import functools

import jax
import jax.numpy as jnp
from jax import lax
from jax.experimental import pallas as pl
from jax.experimental.pallas import tpu as pltpu
from jax.experimental.pallas import tpu_sc as plsc

_VOCAB = 10000
_EMBED = 16
_BATCH = 16384
_MAXLEN = 500

_NC = 2
_NS = 16
_NW = _NC * _NS
_ROWS_PER_W = _BATCH // _NW
_C = 8
_NCHUNK = _ROWS_PER_W // _C
_LPAD = 512
_NSTEP = _LPAD // 16
_VPAD = _VOCAB + 16
_NPAIR = _EMBED // 2


def _sc_pool(idx_hbm, tbl_hbm, part_hbm, idx_a, idx_b, tbl_v, acc_v,
             sem_a, sem_b):
    wid = lax.axis_index("s") * _NC + lax.axis_index("c")
    row0 = wid * _ROWS_PER_W

    pltpu.sync_copy(tbl_hbm, tbl_v)

    cvecs = [jnp.full((16,), c, jnp.int32) for c in range(_NPAIR)]
    himask = jnp.full((16,), -65536, jnp.int32)
    zero = jnp.zeros((16,), jnp.float32)

    def load_idx(ci, buf, sem):
        r0 = row0 + ci * _C
        pltpu.async_copy(idx_hbm.at[pl.ds(r0, _C)], buf, sem)

    def drain_idx(ci, buf, sem):
        r0 = row0 + ci * _C
        pltpu.make_async_copy(idx_hbm.at[pl.ds(r0, _C)], buf, sem).wait()

    def compute(ci, buf):
        r0 = row0 + ci * _C
        for r in range(_C):
            def step(t, accs):
                out = list(accs)
                for u in range(4):
                    idx16 = buf[r, pl.ds((t * 4 + u) * 16, 16)]
                    for c in range(_NPAIR):
                        w = plsc.load_gather(tbl_v, [idx16 + (c * _VPAD)])
                        lo = plsc.bitcast(jnp.left_shift(w, 16), jnp.float32)
                        hi = plsc.bitcast(jnp.bitwise_and(w, himask),
                                          jnp.float32)
                        out[2 * c] = out[2 * c] + lo
                        out[2 * c + 1] = out[2 * c + 1] + hi
                return tuple(out)

            accs = lax.fori_loop(0, _NSTEP // 4, step, (zero,) * _EMBED)
            for c2 in range(_EMBED):
                acc_v[r, pl.ds(c2 * 16, 16)] = accs[c2]
        pltpu.sync_copy(acc_v, part_hbm.at[pl.ds(r0, _C)])

    pltpu.sync_copy(idx_hbm.at[pl.ds(row0, _C)], idx_a)

    def pair_body(k, carry):
        ci0 = 2 * k
        ci1 = 2 * k + 1
        load_idx(ci1, idx_b, sem_b)
        compute(ci0, idx_a)
        drain_idx(ci1, idx_b, sem_b)

        @pl.when(k < _NCHUNK // 2 - 1)
        def _():
            load_idx(ci1 + 1, idx_a, sem_a)

        compute(ci1, idx_b)

        @pl.when(k < _NCHUNK // 2 - 1)
        def _():
            drain_idx(ci1 + 1, idx_a, sem_a)

        return carry

    lax.fori_loop(0, _NCHUNK // 2, pair_body, 0)


def _make_sc_pool():
    mesh = plsc.VectorSubcoreMesh(core_axis_name="c", subcore_axis_name="s")
    return functools.partial(
        pl.kernel,
        mesh=mesh,
        compiler_params=pltpu.CompilerParams(use_tc_tiling_on_sc=False, needs_layout_passes=False),
        out_type=jax.ShapeDtypeStruct((_BATCH, _EMBED * 16), jnp.float32),
        scratch_types=[
            pltpu.VMEM((_C, _LPAD), jnp.int32),
            pltpu.VMEM((_C, _LPAD), jnp.int32),
            pltpu.VMEM((_NPAIR * _VPAD,), jnp.int32),
            pltpu.VMEM((_C, _EMBED * 16), jnp.float32),
            pltpu.SemaphoreType.DMA,
            pltpu.SemaphoreType.DMA,
        ],
    )(_sc_pool)


def _mlp_body(part_ref, fold_ref, w1_ref, b1_ref, w2r_ref, b2_ref, out_ref):
    pooled = jnp.dot(part_ref[...], fold_ref[...],
                     preferred_element_type=jnp.float32) * (1.0 / _MAXLEN)
    h = jnp.dot(pooled, w1_ref[...], preferred_element_type=jnp.float32)
    h = jnp.maximum(h + b1_ref[...], 0.0)
    z = jnp.sum(h * w2r_ref[...], axis=1, keepdims=True) + b2_ref[...]
    out_ref[...] = 1.0 / (1.0 + jnp.exp(-z))


def _mlp(part, fold, W1, b1, W2, b2):
    blk = 2048
    grid = (_BATCH // blk,)
    return pl.pallas_call(
        _mlp_body,
        grid=grid,
        in_specs=[
            pl.BlockSpec((blk, _EMBED * 16), lambda i: (i, 0)),
            pl.BlockSpec((_EMBED * 16, _EMBED), lambda i: (0, 0)),
            pl.BlockSpec((_EMBED, _EMBED), lambda i: (0, 0)),
            pl.BlockSpec((1, _EMBED), lambda i: (0, 0)),
            pl.BlockSpec((1, _EMBED), lambda i: (0, 0)),
            pl.BlockSpec((1, 1), lambda i: (0, 0)),
        ],
        out_specs=pl.BlockSpec((blk, 1), lambda i: (i, 0)),
        out_shape=jax.ShapeDtypeStruct((_BATCH, 1), jnp.float32),
    )(part, fold, W1, b1.reshape(1, _EMBED), W2.reshape(1, _EMBED),
      b2.reshape(1, 1))


def kernel(inputs, table, W1, b1, W2, b2):
    idx = jnp.pad(inputs.astype(jnp.int32), ((0, 0), (0, _LPAD - _MAXLEN)),
                  constant_values=_VOCAB)
    tpad = jnp.concatenate(
        [table, jnp.zeros((_VPAD - _VOCAB, _EMBED), jnp.float32)], axis=0)
    bits = jax.lax.bitcast_convert_type(
        tpad.astype(jnp.bfloat16), jnp.uint16).astype(jnp.uint32)
    packed = bits[:, 0::2] | (bits[:, 1::2] << 16)
    tblT = jax.lax.bitcast_convert_type(packed.T, jnp.int32).reshape(-1)
    part = _make_sc_pool()(idx, tblT)
    fold = jnp.repeat(jnp.eye(_EMBED, dtype=jnp.float32), 16, axis=0)
    return _mlp(part, fold, W1, b1, W2, b2)
